# Initial kernel scaffold; baseline (speedup 1.0000x reference)
import jax
import jax.numpy as jnp
from jax.experimental import pallas as pl


def kernel(bot_mlp_input, cat_features, embedding_table, bot_W0, bot_b0, bot_W1, bot_b1, bot_W2, bot_b2, top_W0, top_b0, top_W1, top_b1, top_W2, top_b2, top_W3, top_b3, top_W4, top_b4):
    raise NotImplementedError("write your pallas kernel here")


# speedup vs baseline: 8.3985x; 8.3985x over previous
import functools

import numpy as np
import jax
import jax.numpy as jnp
from jax import lax
from jax.experimental import pallas as pl
from jax.experimental.pallas import tpu as pltpu
from jax.experimental.pallas import tpu_sc as plsc

_BATCH = 4096
_D = 128
_NSP = 26
_VOCAB = 100000
_NIDX = _BATCH * _NSP
_WIN = 128
_NWIN = _NIDX // _WIN
_F = _NSP + 1
_R = 256


def _sc_gather(table, idx):
    mesh = plsc.VectorSubcoreMesh(core_axis_name="core",
                                  subcore_axis_name="subcore")
    idx2 = idx.reshape(1, _NIDX)

    @functools.partial(
        pl.kernel,
        out_type=jax.ShapeDtypeStruct((_NIDX, _D), jnp.float32),
        mesh=mesh)
    def gather_kernel(x_hbm, i_hbm, o_hbm):
        def body(i_vmem, o_vmem):
            pltpu.sync_copy(x_hbm.at[i_vmem.at[0]], o_vmem)

        pltpu.emit_pipeline(
            body,
            grid=(_NWIN,),
            in_specs=[pl.BlockSpec((1, _WIN), index_map=lambda i: (0, i))],
            out_specs=[pl.BlockSpec((_WIN, _D), index_map=lambda i: (i, 0))],
            core_axis_name=("core", "subcore"),
            dimension_semantics=(pltpu.PARALLEL,),
        )(i_hbm, o_hbm)

    return gather_kernel(table, idx2)


def _dense_body(x_ref, emb_ref, bw0, bb0, bw1, bb1, bw2, bb2,
                w0a, w0s3, tb0, tw1, tb1, tw2, tb2, tw3, tb3, tw4, tb4,
                out_ref):
    f32 = jnp.float32
    h = x_ref[...]
    h = jnp.maximum(jnp.dot(h, bw0[...], preferred_element_type=f32) + bb0[...], 0.0)
    h = jnp.maximum(jnp.dot(h, bw1[...], preferred_element_type=f32) + bb1[...], 0.0)
    bot = jnp.maximum(jnp.dot(h, bw2[...], preferred_element_type=f32) + bb2[...], 0.0)
    emb = emb_ref[...]
    stack = jnp.concatenate(
        [bot.reshape(_R, 1, _D), emb.reshape(_R, _NSP, _D)], axis=1)
    xact = lax.dot_general(stack, stack, (((2,), (2,)), ((0,), (0,))),
                           preferred_element_type=f32)
    acc = jnp.dot(bot, w0a[...], preferred_element_type=f32) + tb0[...]
    for u in range(_F):
        acc = acc + jnp.dot(xact[:, u, :], w0s3[u], preferred_element_type=f32)
    h = jnp.maximum(acc, 0.0)
    h = jnp.maximum(jnp.dot(h, tw1[...], preferred_element_type=f32) + tb1[...], 0.0)
    h = jnp.maximum(jnp.dot(h, tw2[...], preferred_element_type=f32) + tb2[...], 0.0)
    h = jnp.maximum(jnp.dot(h, tw3[...], preferred_element_type=f32) + tb3[...], 0.0)
    out_ref[...] = jnp.dot(h, tw4[...], preferred_element_type=f32) + tb4[...]


def _dense(x, emb2, *ws):
    specs = [pl.BlockSpec((_R, 13), lambda i: (i, 0)),
             pl.BlockSpec((_R, _NSP * _D), lambda i: (i, 0))]
    for w in ws:
        specs.append(pl.BlockSpec(w.shape, lambda i, n=w.ndim: (0,) * n))
    return pl.pallas_call(
        _dense_body,
        grid=(_BATCH // _R,),
        in_specs=specs,
        out_specs=pl.BlockSpec((_R, 1), lambda i: (i, 0)),
        out_shape=jax.ShapeDtypeStruct((_BATCH, 1), jnp.float32),
    )(x, emb2, *ws)


_TRIU_MAP = np.full((_F, _F), _F * (_F + 1) // 2, np.int32)
_TRIU_MAP[np.triu_indices(_F)] = np.arange(_F * (_F + 1) // 2)


def kernel(bot_mlp_input, cat_features, embedding_table,
           bot_W0, bot_b0, bot_W1, bot_b1, bot_W2, bot_b2,
           top_W0, top_b0, top_W1, top_b1, top_W2, top_b2,
           top_W3, top_b3, top_W4, top_b4):
    offs = jnp.arange(_NSP, dtype=jnp.int32) * _VOCAB
    idx = (cat_features.astype(jnp.int32) + offs[None, :]).reshape(-1)
    emb_flat = _sc_gather(embedding_table, idx)
    emb2 = emb_flat.reshape(_BATCH, _NSP * _D)

    n_out = top_W0.shape[1]
    w0a = top_W0[:_D]
    w0pad = jnp.concatenate(
        [top_W0[_D:], jnp.zeros((1, n_out), jnp.float32)], axis=0)
    w0s3 = w0pad[jnp.asarray(_TRIU_MAP.reshape(-1))].reshape(_F, _F, n_out)

    row = lambda b: b.reshape(1, -1)
    return _dense(bot_mlp_input, emb2,
                  bot_W0, row(bot_b0), bot_W1, row(bot_b1), bot_W2, row(bot_b2),
                  w0a, w0s3, row(top_b0), top_W1, row(top_b1),
                  top_W2, row(top_b2), top_W3, row(top_b3),
                  top_W4, row(top_b4))


# speedup vs baseline: 10.1859x; 1.2128x over previous
import functools

import numpy as np
import jax
import jax.numpy as jnp
from jax import lax
from jax.experimental import pallas as pl
from jax.experimental.pallas import tpu as pltpu
from jax.experimental.pallas import tpu_sc as plsc

_BATCH = 4096
_D = 128
_NSP = 26
_VOCAB = 100000
_NIDX = _BATCH * _NSP
_WIN = 128
_NWIN = _NIDX // _WIN
_F = _NSP + 1
_FP = 32
_R = 256


def _sc_gather(table, idx):
    mesh = plsc.VectorSubcoreMesh(core_axis_name="core",
                                  subcore_axis_name="subcore")
    idx2 = idx.reshape(1, _NIDX)

    @functools.partial(
        pl.kernel,
        out_type=jax.ShapeDtypeStruct((_NIDX, _D), jnp.float32),
        mesh=mesh)
    def gather_kernel(x_hbm, i_hbm, o_hbm):
        def body(i_vmem, o_vmem):
            pltpu.sync_copy(x_hbm.at[i_vmem.at[0]], o_vmem)

        pltpu.emit_pipeline(
            body,
            grid=(_NWIN,),
            in_specs=[pl.BlockSpec((1, _WIN), index_map=lambda i: (0, i))],
            out_specs=[pl.BlockSpec((_WIN, _D), index_map=lambda i: (i, 0))],
            core_axis_name=("core", "subcore"),
            dimension_semantics=(pltpu.PARALLEL,),
        )(i_hbm, o_hbm)

    return gather_kernel(table, idx2)


def _dense_body(x_ref, emb_ref, bw0, bb0, bw1, bb1, bw2, bb2,
                w0a, w0s3, tb0, tw1, tb1, tw2, tb2, tw3, tb3, tw4, tb4,
                out_ref):
    f32 = jnp.float32
    bf = jnp.bfloat16
    h = x_ref[...].astype(bf)
    h = jnp.maximum(jnp.dot(h, bw0[...], preferred_element_type=f32) + bb0[...], 0.0)
    h = jnp.maximum(jnp.dot(h.astype(bf), bw1[...], preferred_element_type=f32) + bb1[...], 0.0)
    bot = jnp.maximum(jnp.dot(h.astype(bf), bw2[...], preferred_element_type=f32) + bb2[...], 0.0)
    botb = bot.astype(bf)
    emb = emb_ref[...].astype(bf)
    stack = jnp.concatenate(
        [botb.reshape(_R, 1, _D), emb.reshape(_R, _NSP, _D),
         jnp.zeros((_R, _FP - _F, _D), bf)], axis=1)
    xact = lax.dot_general(stack, stack, (((2,), (2,)), ((0,), (0,))),
                           preferred_element_type=f32)
    xflat = xact.astype(bf).reshape(_R, _FP * _FP)
    acc = (jnp.dot(botb, w0a[...], preferred_element_type=f32)
           + jnp.dot(xflat, w0s3[...], preferred_element_type=f32) + tb0[...])
    h = jnp.maximum(acc, 0.0)
    h = jnp.maximum(jnp.dot(h.astype(bf), tw1[...], preferred_element_type=f32) + tb1[...], 0.0)
    h = jnp.maximum(jnp.dot(h.astype(bf), tw2[...], preferred_element_type=f32) + tb2[...], 0.0)
    h = jnp.maximum(jnp.dot(h.astype(bf), tw3[...], preferred_element_type=f32) + tb3[...], 0.0)
    out_ref[...] = jnp.dot(h.astype(bf), tw4[...], preferred_element_type=f32) + tb4[...]


def _dense(x, emb2, *ws):
    specs = [pl.BlockSpec((_R, 13), lambda i: (i, 0)),
             pl.BlockSpec((_R, _NSP * _D), lambda i: (i, 0))]
    for w in ws:
        specs.append(pl.BlockSpec(w.shape, lambda i, n=w.ndim: (0,) * n))
    return pl.pallas_call(
        _dense_body,
        grid=(_BATCH // _R,),
        in_specs=specs,
        out_specs=pl.BlockSpec((_R, 1), lambda i: (i, 0)),
        out_shape=jax.ShapeDtypeStruct((_BATCH, 1), jnp.float32),
    )(x, emb2, *ws)


_TRIU_MAP = np.full((_FP, _FP), _F * (_F + 1) // 2, np.int32)
_TRIU_MAP[np.triu_indices(_F)] = np.arange(_F * (_F + 1) // 2)


def kernel(bot_mlp_input, cat_features, embedding_table,
           bot_W0, bot_b0, bot_W1, bot_b1, bot_W2, bot_b2,
           top_W0, top_b0, top_W1, top_b1, top_W2, top_b2,
           top_W3, top_b3, top_W4, top_b4):
    offs = jnp.arange(_NSP, dtype=jnp.int32) * _VOCAB
    idx = (cat_features.astype(jnp.int32) + offs[None, :]).reshape(-1)
    emb_flat = _sc_gather(embedding_table, idx)
    emb2 = emb_flat.reshape(_BATCH, _NSP * _D)

    n_out = top_W0.shape[1]
    bf = jnp.bfloat16
    w0a = top_W0[:_D].astype(bf)
    w0pad = jnp.concatenate(
        [top_W0[_D:], jnp.zeros((1, n_out), jnp.float32)], axis=0).astype(bf)
    w0s3 = w0pad[jnp.asarray(_TRIU_MAP.reshape(-1))]

    row = lambda b: b.reshape(1, -1)
    return _dense(bot_mlp_input, emb2,
                  bot_W0.astype(bf), row(bot_b0), bot_W1.astype(bf),
                  row(bot_b1), bot_W2.astype(bf), row(bot_b2),
                  w0a, w0s3, row(top_b0), top_W1.astype(bf), row(top_b1),
                  top_W2.astype(bf), row(top_b2), top_W3.astype(bf),
                  row(top_b3), top_W4.astype(bf), row(top_b4))


# speedup vs baseline: 12.7791x; 1.2546x over previous
import functools

import numpy as np
import jax
import jax.numpy as jnp
from jax import lax
from jax.experimental import pallas as pl
from jax.experimental.pallas import tpu as pltpu
from jax.experimental.pallas import tpu_sc as plsc

_BATCH = 4096
_D = 128
_NSP = 26
_VOCAB = 100000
_NIDX = _BATCH * _NSP
_WIN = 128
_NWIN = _NIDX // _WIN
_F = _NSP + 1
_FP = 32
_R = 256


def _sc_gather(table, idx):
    mesh = plsc.VectorSubcoreMesh(core_axis_name="core",
                                  subcore_axis_name="subcore")
    idx2 = idx.reshape(1, _NIDX)

    @functools.partial(
        pl.kernel,
        out_type=jax.ShapeDtypeStruct((_NIDX, _D), jnp.float32),
        mesh=mesh)
    def gather_kernel(x_hbm, i_hbm, o_hbm):
        def body(i_vmem, o_vmem):
            pltpu.sync_copy(x_hbm.at[i_vmem.at[0]], o_vmem)

        pltpu.emit_pipeline(
            body,
            grid=(_NWIN,),
            in_specs=[pl.BlockSpec((1, _WIN), index_map=lambda i: (0, i))],
            out_specs=[pl.BlockSpec((_WIN, _D), index_map=lambda i: (i, 0))],
            core_axis_name=("core", "subcore"),
            dimension_semantics=(pltpu.PARALLEL,),
        )(i_hbm, o_hbm)

    return gather_kernel(table, idx2)


def _dense_body(x_ref, emb_ref, bw0, bb0, bw1, bb1, bw2, bb2,
                w0a, w0s3, tb0, tw1, tb1, tw2, tb2, tw3, tb3, tw4, tb4,
                out_ref):
    f32 = jnp.float32
    bf = jnp.bfloat16
    h = x_ref[...].astype(bf)
    h = jnp.maximum(jnp.dot(h, bw0[...], preferred_element_type=f32) + bb0[...], 0.0)
    h = jnp.maximum(jnp.dot(h.astype(bf), bw1[...], preferred_element_type=f32) + bb1[...], 0.0)
    bot = jnp.maximum(jnp.dot(h.astype(bf), bw2[...], preferred_element_type=f32) + bb2[...], 0.0)
    botb = bot.astype(bf)
    emb = emb_ref[...].astype(bf)
    stack = jnp.concatenate(
        [botb.reshape(_R, 1, _D), emb.reshape(_R, _NSP, _D),
         jnp.zeros((_R, _FP - _F, _D), bf)], axis=1)
    xact = lax.dot_general(stack, stack, (((2,), (2,)), ((0,), (0,))),
                           preferred_element_type=f32)
    xflat = xact.astype(bf).reshape(_R, _FP * _FP)
    acc = (jnp.dot(botb, w0a[...], preferred_element_type=f32)
           + jnp.dot(xflat, w0s3[...], preferred_element_type=f32) + tb0[...])
    h = jnp.maximum(acc, 0.0)
    h = jnp.maximum(jnp.dot(h.astype(bf), tw1[...], preferred_element_type=f32) + tb1[...], 0.0)
    h = jnp.maximum(jnp.dot(h.astype(bf), tw2[...], preferred_element_type=f32) + tb2[...], 0.0)
    h = jnp.maximum(jnp.dot(h.astype(bf), tw3[...], preferred_element_type=f32) + tb3[...], 0.0)
    out_ref[...] = jnp.dot(h.astype(bf), tw4[...], preferred_element_type=f32) + tb4[...]


def _dense(x, emb2, *ws):
    specs = [pl.BlockSpec((_R, 13), lambda i: (i, 0)),
             pl.BlockSpec((_R * _NSP, _D), lambda i: (i, 0))]
    for w in ws:
        specs.append(pl.BlockSpec(w.shape, lambda i, n=w.ndim: (0,) * n))
    return pl.pallas_call(
        _dense_body,
        grid=(_BATCH // _R,),
        in_specs=specs,
        out_specs=pl.BlockSpec((_R, 1), lambda i: (i, 0)),
        out_shape=jax.ShapeDtypeStruct((_BATCH, 1), jnp.float32),
    )(x, emb2, *ws)


_TRIU_MAP = np.full((_FP, _FP), _F * (_F + 1) // 2, np.int32)
_TRIU_MAP[np.triu_indices(_F)] = np.arange(_F * (_F + 1) // 2)


def kernel(bot_mlp_input, cat_features, embedding_table,
           bot_W0, bot_b0, bot_W1, bot_b1, bot_W2, bot_b2,
           top_W0, top_b0, top_W1, top_b1, top_W2, top_b2,
           top_W3, top_b3, top_W4, top_b4):
    offs = jnp.arange(_NSP, dtype=jnp.int32) * _VOCAB
    idx = (cat_features.astype(jnp.int32) + offs[None, :]).reshape(-1)
    emb_flat = _sc_gather(embedding_table, idx)

    n_out = top_W0.shape[1]
    bf = jnp.bfloat16
    w0a = top_W0[:_D].astype(bf)
    w0pad = jnp.concatenate(
        [top_W0[_D:], jnp.zeros((1, n_out), jnp.float32)], axis=0).astype(bf)
    w0s3 = w0pad[jnp.asarray(_TRIU_MAP.reshape(-1))]

    row = lambda b: b.reshape(1, -1)
    return _dense(bot_mlp_input, emb_flat,
                  bot_W0.astype(bf), row(bot_b0), bot_W1.astype(bf),
                  row(bot_b1), bot_W2.astype(bf), row(bot_b2),
                  w0a, w0s3, row(top_b0), top_W1.astype(bf), row(top_b1),
                  top_W2.astype(bf), row(top_b2), top_W3.astype(bf),
                  row(top_b3), top_W4.astype(bf), row(top_b4))


# speedup vs baseline: 14.2763x; 1.1172x over previous
import functools

import numpy as np
import jax
import jax.numpy as jnp
from jax import lax
from jax.experimental import pallas as pl
from jax.experimental.pallas import tpu as pltpu
from jax.experimental.pallas import tpu_sc as plsc

_BATCH = 4096
_D = 128
_NSP = 26
_VOCAB = 100000
_NIDX = _BATCH * _NSP
_WIN = 128
_NWIN = _NIDX // _WIN
_F = _NSP + 1
_FP = 32
_R = 256


def _sc_gather(table, idx):
    n = idx.shape[0]
    nwin = n // _WIN
    mesh = plsc.VectorSubcoreMesh(core_axis_name="core",
                                  subcore_axis_name="subcore")
    idx2 = idx.reshape(1, n)

    @functools.partial(
        pl.kernel,
        out_type=jax.ShapeDtypeStruct((n, _D), jnp.float32),
        mesh=mesh)
    def gather_kernel(x_hbm, i_hbm, o_hbm):
        def body(i_vmem, o_vmem):
            pltpu.sync_copy(x_hbm.at[i_vmem.at[0]], o_vmem)

        pltpu.emit_pipeline(
            body,
            grid=(nwin,),
            in_specs=[pl.BlockSpec((1, _WIN), index_map=lambda i: (0, i))],
            out_specs=[pl.BlockSpec((_WIN, _D), index_map=lambda i: (i, 0))],
            core_axis_name=("core", "subcore"),
            dimension_semantics=(pltpu.PARALLEL,),
        )(i_hbm, o_hbm)

    return gather_kernel(table, idx2)


def _dense_body(x_ref, emb_ref, bw0, bb0, bw1, bb1, bw2, bb2,
                w0a, w0s3, tb0, tw1, tb1, tw2, tb2, tw3, tb3, tw4, tb4,
                out_ref):
    f32 = jnp.float32
    bf = jnp.bfloat16
    h = x_ref[...].astype(bf)
    h = jnp.maximum(jnp.dot(h, bw0[...], preferred_element_type=f32) + bb0[...], 0.0)
    h = jnp.maximum(jnp.dot(h.astype(bf), bw1[...], preferred_element_type=f32) + bb1[...], 0.0)
    bot = jnp.maximum(jnp.dot(h.astype(bf), bw2[...], preferred_element_type=f32) + bb2[...], 0.0)
    botb = bot.astype(bf)
    emb = emb_ref[...].astype(bf)
    stack = jnp.concatenate(
        [botb.reshape(_R, 1, _D), emb.reshape(_R, _NSP, _D),
         jnp.zeros((_R, _FP - _F, _D), bf)], axis=1)
    xact = lax.dot_general(stack, stack, (((2,), (2,)), ((0,), (0,))),
                           preferred_element_type=f32)
    xflat = xact.astype(bf).reshape(_R, _FP * _FP)
    acc = (jnp.dot(botb, w0a[...], preferred_element_type=f32)
           + jnp.dot(xflat, w0s3[...], preferred_element_type=f32) + tb0[...])
    h = jnp.maximum(acc, 0.0)
    h = jnp.maximum(jnp.dot(h.astype(bf), tw1[...], preferred_element_type=f32) + tb1[...], 0.0)
    h = jnp.maximum(jnp.dot(h.astype(bf), tw2[...], preferred_element_type=f32) + tb2[...], 0.0)
    h = jnp.maximum(jnp.dot(h.astype(bf), tw3[...], preferred_element_type=f32) + tb3[...], 0.0)
    out_ref[...] = jnp.dot(h.astype(bf), tw4[...], preferred_element_type=f32) + tb4[...]


def _dense(x, emb2, *ws):
    nb = x.shape[0]
    specs = [pl.BlockSpec((_R, 13), lambda i: (i, 0)),
             pl.BlockSpec((_R * _NSP, _D), lambda i: (i, 0))]
    for w in ws:
        specs.append(pl.BlockSpec(w.shape, lambda i, n=w.ndim: (0,) * n))
    return pl.pallas_call(
        _dense_body,
        grid=(nb // _R,),
        in_specs=specs,
        out_specs=pl.BlockSpec((_R, 1), lambda i: (i, 0)),
        out_shape=jax.ShapeDtypeStruct((nb, 1), jnp.float32),
    )(x, emb2, *ws)


_TRIU_MAP = np.full((_FP, _FP), _F * (_F + 1) // 2, np.int32)
_TRIU_MAP[np.triu_indices(_F)] = np.arange(_F * (_F + 1) // 2)


def kernel(bot_mlp_input, cat_features, embedding_table,
           bot_W0, bot_b0, bot_W1, bot_b1, bot_W2, bot_b2,
           top_W0, top_b0, top_W1, top_b1, top_W2, top_b2,
           top_W3, top_b3, top_W4, top_b4):
    offs = jnp.arange(_NSP, dtype=jnp.int32) * _VOCAB
    idx = (cat_features.astype(jnp.int32) + offs[None, :]).reshape(-1)

    n_out = top_W0.shape[1]
    bf = jnp.bfloat16
    w0a = top_W0[:_D].astype(bf)
    w0pad = jnp.concatenate(
        [top_W0[_D:], jnp.zeros((1, n_out), jnp.float32)], axis=0).astype(bf)
    w0s3 = w0pad[jnp.asarray(_TRIU_MAP.reshape(-1))]

    row = lambda b: b.reshape(1, -1)
    ws = (bot_W0.astype(bf), row(bot_b0), bot_W1.astype(bf),
          row(bot_b1), bot_W2.astype(bf), row(bot_b2),
          w0a, w0s3, row(top_b0), top_W1.astype(bf), row(top_b1),
          top_W2.astype(bf), row(top_b2), top_W3.astype(bf),
          row(top_b3), top_W4.astype(bf), row(top_b4))

    half = _BATCH // 2
    outs = []
    for k in range(2):
        idx_k = lax.dynamic_slice_in_dim(idx, k * half * _NSP, half * _NSP)
        emb_k = _sc_gather(embedding_table, idx_k)
        x_k = lax.dynamic_slice_in_dim(bot_mlp_input, k * half, half)
        outs.append(_dense(x_k, emb_k, *ws))
    return jnp.concatenate(outs, axis=0)
